# baseline (speedup 1.0000x reference)
import functools

import jax
import jax.numpy as jnp
from jax import lax
from jax.experimental import pallas as pl
from jax.experimental.pallas import tpu as pltpu
from jax.experimental.pallas import tpu_sc as plsc

N_CODE = 1024
DIM = 64
TOK = 1024
NB = 16
GB = 4
T = GB * TOK
NTOK = NB * TOK


def _argmin_body(z_ref, w_ref, idx_ref, loss_ref):
    b = pl.program_id(0)
    w = w_ref[...]
    zt = jnp.concatenate([z_ref[k].T for k in range(GB)], axis=0)
    zn = jnp.sum(zt * zt, axis=1, keepdims=True)
    wn = jnp.sum(w * w, axis=1)
    mm2 = jax.lax.dot_general(zt + zt, w, (((1,), (1,)), ((), ())),
                              preferred_element_type=jnp.float32)
    dist = (zn + wn) - mm2
    m = jnp.min(dist, axis=1, keepdims=True)
    iota_j = jax.lax.broadcasted_iota(jnp.int32, dist.shape, 1)
    idx = jnp.min(jnp.where(dist == m, iota_j, N_CODE), axis=1)
    for k in range(GB):
        idx_ref[k, 0, :] = idx[k * TOK:(k + 1) * TOK]
    part = jnp.sum(m, axis=(0, 1), keepdims=True)

    @pl.when(b == 0)
    def _init():
        loss_ref[...] = jnp.zeros((1, 1), jnp.float32)

    loss_ref[...] += part

    @pl.when(b == NB // GB - 1)
    def _fin():
        loss_ref[...] = loss_ref[...] / (NB * TOK * DIM)


_SC_INFO = plsc.get_sparse_core_info()
_NC = _SC_INFO.num_cores
_NS = _SC_INFO.num_subcores
_NW = _NC * _NS
_B_PER_W = NTOK // _NW


def _sc_gather(idx_flat, W128):
    mesh = plsc.VectorSubcoreMesh(core_axis_name="c", subcore_axis_name="s")

    @functools.partial(
        pl.kernel, mesh=mesh,
        out_type=jax.ShapeDtypeStruct((NTOK, 2 * DIM), jnp.float32),
        scratch_types=[
            pltpu.VMEM((_B_PER_W,), jnp.int32),
            pltpu.VMEM((_B_PER_W, 2 * DIM), jnp.float32),
            pltpu.SemaphoreType.DMA,
        ],
    )
    def k(idx_hbm, table_hbm, out_hbm, idx_v, rows_v, sem):
        wid = lax.axis_index("s") * _NC + lax.axis_index("c")
        base = wid * _B_PER_W
        pltpu.sync_copy(idx_hbm.at[pl.ds(base, _B_PER_W)], idx_v)
        pltpu.async_copy(table_hbm.at[idx_v], rows_v, sem).wait()
        pltpu.sync_copy(rows_v, out_hbm.at[pl.ds(base, _B_PER_W)])

    return k(idx_flat, W128)


def _tr_body(rows_ref, out_ref):
    for k in range(GB):
        out_ref[k] = rows_ref[0, k * TOK:(k + 1) * TOK, :DIM].T


def kernel(z, W):
    B, C, H, Wd = z.shape
    z3 = z.reshape(B, C, H * Wd)
    idx3, loss = pl.pallas_call(
        _argmin_body,
        grid=(B // GB,),
        in_specs=[
            pl.BlockSpec((GB, C, H * Wd), lambda b: (b, 0, 0)),
            pl.BlockSpec((N_CODE, DIM), lambda b: (0, 0)),
        ],
        out_specs=[
            pl.BlockSpec((GB, 1, H * Wd), lambda b: (b, 0, 0)),
            pl.BlockSpec((1, 1), lambda b: (0, 0)),
        ],
        out_shape=[
            jax.ShapeDtypeStruct((B, 1, H * Wd), jnp.int32),
            jax.ShapeDtypeStruct((1, 1), jnp.float32),
        ],
    )(z3, W)
    idx_flat = idx3.reshape(NTOK)
    W128 = jnp.pad(W, ((0, 0), (0, 2 * DIM - DIM)))
    rows = _sc_gather(idx_flat, W128)
    rows3 = rows.reshape(1, NTOK, 2 * DIM)
    zq3 = pl.pallas_call(
        _tr_body,
        grid=(B // GB,),
        in_specs=[pl.BlockSpec((1, T, 2 * DIM), lambda b: (0, b, 0))],
        out_specs=pl.BlockSpec((GB, DIM, H * Wd), lambda b: (b, 0, 0)),
        out_shape=jax.ShapeDtypeStruct((B, DIM, H * Wd), jnp.float32),
    )(rows3)
    z_q = zq3.reshape(B, C, H, Wd)
    codebook_loss = loss.reshape(())
    commitment_loss = 0.25 * codebook_loss
    min_encoding_indices = idx3.reshape(B, H, Wd)
    return (z_q, codebook_loss, commitment_loss, min_encoding_indices)


# speedup vs baseline: 1.5468x; 1.5468x over previous
import jax
import jax.numpy as jnp
from jax.experimental import pallas as pl

N_CODE = 1024
DIM = 64
TOK = 1024
NB = 16
GB = 4
T = GB * TOK


def _vq_body(z_ref, w_ref, zq_ref, idx_ref, loss_ref):
    b = pl.program_id(0)
    w = w_ref[...]
    zt = jnp.concatenate([z_ref[k].T for k in range(GB)], axis=0)
    zn = jnp.sum(zt * zt, axis=1, keepdims=True)
    wn = jnp.sum(w * w, axis=1)
    mm2 = jax.lax.dot_general(zt + zt, w, (((1,), (1,)), ((), ())),
                              preferred_element_type=jnp.float32)
    dist = (zn + wn) - mm2
    m = jnp.min(dist, axis=1, keepdims=True)
    jf = jax.lax.broadcasted_iota(jnp.int32, (1, N_CODE), 1).astype(jnp.float32)
    idx_f = jnp.min((dist - m) * jnp.float32(2.0 ** 37) + jf, axis=1)
    idx = idx_f.astype(jnp.int32)
    e = (jax.lax.broadcasted_iota(jnp.int32, (N_CODE, T), 0)
         == idx[None, :]).astype(jnp.bfloat16)
    zq_t = jax.lax.dot_general(w.astype(jnp.bfloat16), e,
                               (((0,), (0,)), ((), ())),
                               preferred_element_type=jnp.float32)
    for k in range(GB):
        idx_ref[k, 0, :] = idx[k * TOK:(k + 1) * TOK]
        zq_ref[k] = zq_t[:, k * TOK:(k + 1) * TOK]
    part = jnp.sum(m, axis=(0, 1), keepdims=True)

    @pl.when(b == 0)
    def _init():
        loss_ref[...] = jnp.zeros((1, 1), jnp.float32)

    loss_ref[...] += part

    @pl.when(b == NB // GB - 1)
    def _fin():
        loss_ref[...] = loss_ref[...] / (NB * TOK * DIM)


def kernel(z, W):
    B, C, H, Wd = z.shape
    z3 = z.reshape(B, C, H * Wd)
    zq3, idx3, loss = pl.pallas_call(
        _vq_body,
        grid=(B // GB,),
        in_specs=[
            pl.BlockSpec((GB, C, H * Wd), lambda b: (b, 0, 0)),
            pl.BlockSpec((N_CODE, DIM), lambda b: (0, 0)),
        ],
        out_specs=[
            pl.BlockSpec((GB, C, H * Wd), lambda b: (b, 0, 0)),
            pl.BlockSpec((GB, 1, H * Wd), lambda b: (b, 0, 0)),
            pl.BlockSpec((1, 1), lambda b: (0, 0)),
        ],
        out_shape=[
            jax.ShapeDtypeStruct((B, C, H * Wd), jnp.float32),
            jax.ShapeDtypeStruct((B, 1, H * Wd), jnp.int32),
            jax.ShapeDtypeStruct((1, 1), jnp.float32),
        ],
    )(z3, W)
    z_q = zq3.reshape(B, C, H, Wd)
    codebook_loss = loss.reshape(())
    commitment_loss = 0.25 * codebook_loss
    min_encoding_indices = idx3.reshape(B, H, Wd)
    return (z_q, codebook_loss, commitment_loss, min_encoding_indices)


# speedup vs baseline: 1.7252x; 1.1153x over previous
import jax
import jax.numpy as jnp
from jax.experimental import pallas as pl

N_CODE = 1024
DIM = 64
TOK = 1024
NB = 16
GB = 4
T = GB * TOK


def _vq_body(z_ref, w_ref, zq_ref, idx_ref, loss_ref):
    b = pl.program_id(0)
    w = w_ref[...]
    zc = jnp.concatenate([z_ref[k] for k in range(GB)], axis=1)
    zn = jnp.sum(zc * zc, axis=0, keepdims=True)
    wn = jnp.sum(w * w, axis=1, keepdims=True)
    mm2 = jax.lax.dot_general(w + w, zc, (((1,), (0,)), ((), ())),
                              preferred_element_type=jnp.float32)
    dist = (zn + wn) - mm2
    m = jnp.min(dist, axis=0, keepdims=True)
    jf = jax.lax.broadcasted_iota(jnp.int32, (N_CODE, 1), 0).astype(jnp.float32)
    idx_f = jnp.min((dist - m) * jnp.float32(2.0 ** 37) + jf, axis=0)
    idx = idx_f.astype(jnp.int32)
    e = (jax.lax.broadcasted_iota(jnp.int32, (N_CODE, T), 0)
         == idx[None, :]).astype(jnp.bfloat16)
    zq_t = jax.lax.dot_general(w.astype(jnp.bfloat16), e,
                               (((0,), (0,)), ((), ())),
                               preferred_element_type=jnp.float32)
    for k in range(GB):
        idx_ref[k, 0, :] = idx[k * TOK:(k + 1) * TOK]
        zq_ref[k] = zq_t[:, k * TOK:(k + 1) * TOK]
    part = jnp.sum(m, axis=(0, 1), keepdims=True)

    @pl.when(b == 0)
    def _init():
        loss_ref[...] = jnp.zeros((1, 1), jnp.float32)

    loss_ref[...] += part

    @pl.when(b == NB // GB - 1)
    def _fin():
        loss_ref[...] = loss_ref[...] / (NB * TOK * DIM)


def kernel(z, W):
    B, C, H, Wd = z.shape
    z3 = z.reshape(B, C, H * Wd)
    zq3, idx3, loss = pl.pallas_call(
        _vq_body,
        grid=(B // GB,),
        in_specs=[
            pl.BlockSpec((GB, C, H * Wd), lambda b: (b, 0, 0)),
            pl.BlockSpec((N_CODE, DIM), lambda b: (0, 0)),
        ],
        out_specs=[
            pl.BlockSpec((GB, C, H * Wd), lambda b: (b, 0, 0)),
            pl.BlockSpec((GB, 1, H * Wd), lambda b: (b, 0, 0)),
            pl.BlockSpec((1, 1), lambda b: (0, 0)),
        ],
        out_shape=[
            jax.ShapeDtypeStruct((B, C, H * Wd), jnp.float32),
            jax.ShapeDtypeStruct((B, 1, H * Wd), jnp.int32),
            jax.ShapeDtypeStruct((1, 1), jnp.float32),
        ],
    )(z3, W)
    z_q = zq3.reshape(B, C, H, Wd)
    codebook_loss = loss.reshape(())
    commitment_loss = 0.25 * codebook_loss
    min_encoding_indices = idx3.reshape(B, H, Wd)
    return (z_q, codebook_loss, commitment_loss, min_encoding_indices)
